# baseline (speedup 1.0000x reference)
import jax
import jax.numpy as jnp
from jax import lax
from jax.experimental import pallas as pl


def _body(uh_ref, xu_ref, xi_ref, av_ref, cut_ref, cie_ref, s_ref, out_ref):
    I, P, Bn = xu_ref.shape
    cu = jnp.dot(cut_ref[...], uh_ref[...], preferred_element_type=jnp.float32)
    yu = xu_ref[...] * cu[None, :, :]
    yi = xi_ref[...] * cie_ref[...]
    t = jnp.dot(s_ref[...], (yu + yi).reshape(I * P, Bn),
                preferred_element_type=jnp.float32)
    out_ref[...] = jnp.where(av_ref[...] != 0, t, jnp.float32(-1e20))


def kernel(x_u, x_i, availability, user_onehot, coef_u, coef_i):
    B, I, P = x_u.shape
    U = coef_u.shape[0]
    IP = I * P

    xu_t = jnp.transpose(x_u, (1, 2, 0))
    xi_t = jnp.transpose(x_i, (1, 2, 0))
    uh_t = jnp.transpose(user_onehot, (1, 2, 0)).reshape(U, B)
    av_t = availability.T.astype(jnp.int8)
    cu_t = coef_u.T
    cie = coef_i[:, :, None]
    jj = jnp.arange(IP, dtype=jnp.int32)
    ii = jnp.arange(I, dtype=jnp.int32)
    s_t = (jj[None, :] // P == ii[:, None]).astype(jnp.float32)

    Bn = 1024
    out_t = pl.pallas_call(
        _body,
        grid=(B // Bn,),
        in_specs=[
            pl.BlockSpec((U, Bn), lambda i: (0, i)),
            pl.BlockSpec((I, P, Bn), lambda i: (0, 0, i)),
            pl.BlockSpec((I, P, Bn), lambda i: (0, 0, i)),
            pl.BlockSpec((I, Bn), lambda i: (0, i)),
            pl.BlockSpec((P, U), lambda i: (0, 0)),
            pl.BlockSpec((I, P, 1), lambda i: (0, 0, 0)),
            pl.BlockSpec((I, IP), lambda i: (0, 0)),
        ],
        out_specs=pl.BlockSpec((I, Bn), lambda i: (0, i)),
        out_shape=jax.ShapeDtypeStruct((I, B), jnp.float32),
    )(uh_t, xu_t, xi_t, av_t, cu_t, cie, s_t)
    return out_t.T
